# baseline (speedup 1.0000x reference)
import functools

import jax
import jax.numpy as jnp
from jax import lax
from jax.experimental import pallas as pl
from jax.experimental.pallas import tpu as pltpu
from jax.experimental.pallas import tpu_sc as plsc

N_EMBD = 256
N_EXPERTS = 8
HIDDEN = 682
HID_PAD = 768
TILE = 256
NW = 32


def _shift_rows(a, k):
    return jnp.pad(a, ((k, 0), (0, 0)))[: a.shape[0], :]


def _shift_lanes(a, k):
    return jnp.pad(a, ((0, 0), (k, 0)))[:, : a.shape[1]]


def _make_plan_kernel(nt):
    n_ent = 2 * nt

    def _plan_kernel(x_ref, wr_ref, pos_ref, wn_ref, seg_ref):
        x = x_ref[...]
        logits_t = lax.dot_general(
            wr_ref[...], x, (((1,), (1,)), ((), ())),
            preferred_element_type=jnp.float32)
        m = jnp.max(logits_t, axis=0, keepdims=True)
        unnorm = jnp.exp(logits_t - m)
        probs = unnorm / jnp.sum(unnorm, axis=0, keepdims=True)
        eidx = lax.broadcasted_iota(jnp.int32, probs.shape, 0)
        p1 = jnp.max(probs, axis=0, keepdims=True)
        i1 = jnp.min(jnp.where(probs == p1, eidx, N_EXPERTS), axis=0,
                     keepdims=True)
        probs_m = jnp.where(eidx == i1, -1.0, probs)
        p2 = jnp.max(probs_m, axis=0, keepdims=True)
        i2 = jnp.min(jnp.where(probs_m == p2, eidx, N_EXPERTS), axis=0,
                     keepdims=True)
        denom = p1 + p2 + 1e-9
        w1n = p1 / denom
        w2n = p2 / denom

        oh1 = (eidx == i1).astype(jnp.float32)
        oh2 = (eidx == i2).astype(jnp.float32)
        oht = jnp.concatenate([oh1, oh2], axis=1)

        inc = oht
        k = 1
        while k < n_ent:
            inc = inc + _shift_lanes(inc, k)
            k *= 2
        excl = inc - oht

        counts_i = inc[:, n_ent - 1:n_ent].astype(jnp.int32)
        padded = ((counts_i + TILE - 1) // TILE) * TILE
        t = padded
        for k in (1, 2, 4):
            t = t + _shift_rows(t, k)
        seg_start = t - padded
        seg_end = t

        rank = jnp.sum(excl * oht, axis=0, keepdims=True)
        base = jnp.sum(oht * seg_start.astype(jnp.float32), axis=0,
                       keepdims=True)
        pos_ref[...] = (rank + base).astype(jnp.int32)
        wn_ref[...] = jnp.concatenate([w1n, w2n], axis=1)
        seg_ref[...] = jnp.concatenate([seg_start, seg_end], axis=1)

    return _plan_kernel


def _make_dispatch_kernel(nt):
    tpw = nt // NW
    assert tpw == 128

    def _dispatch_kernel(pos_hbm, wn_hbm, flat_hbm, xs_hbm, wrow_hbm,
                         idx_v, wnv_v, rows_v, sem):
        cid = lax.axis_index("c")
        sid = lax.axis_index("s")
        wid = sid * 2 + cid

        pltpu.sync_copy(flat_hbm.at[pl.ds(wid * tpw, tpw)], rows_v)
        pltpu.sync_copy(pos_hbm.at[pl.ds(wid, 1)], idx_v.at[pl.ds(0, 1)])
        pltpu.sync_copy(pos_hbm.at[pl.ds(nt // 128 + wid, 1)],
                        idx_v.at[pl.ds(1, 1)])
        pltpu.sync_copy(wn_hbm.at[pl.ds(wid, 1)], wnv_v.at[pl.ds(0, 1)])
        pltpu.sync_copy(wn_hbm.at[pl.ds(nt // 128 + wid, 1)],
                        wnv_v.at[pl.ds(1, 1)])

        c0 = pltpu.async_copy(rows_v, xs_hbm.at[idx_v.at[0]], sem)
        c1 = pltpu.async_copy(rows_v, xs_hbm.at[idx_v.at[1]], sem)
        c0.wait()
        c1.wait()
        pltpu.sync_copy(wnv_v.at[0], wrow_hbm.at[idx_v.at[0]])
        pltpu.sync_copy(wnv_v.at[1], wrow_hbm.at[idx_v.at[1]])

    return _dispatch_kernel


def _expert_kernel(te_ref, xs_ref, wrow_ref, w1_ref, w3_ref, w2_ref,
                   ex_ref):
    xb = xs_ref[...].astype(jnp.bfloat16)
    g = lax.dot_general(
        xb, w1_ref[0], (((1,), (1,)), ((), ())),
        preferred_element_type=jnp.float32)
    u = lax.dot_general(
        xb, w3_ref[0], (((1,), (1,)), ((), ())),
        preferred_element_type=jnp.float32)
    h = (g * jax.nn.sigmoid(g)) * u
    ex = lax.dot_general(
        h.astype(jnp.bfloat16), w2_ref[0], (((1,), (1,)), ((), ())),
        preferred_element_type=jnp.float32)
    ex_ref[...] = ex * wrow_ref[0, 0][:, None]


def _make_combine_kernel(nt):
    tpw = nt // NW
    assert tpw == 128

    def _combine_kernel(pos_hbm, ex_hbm, out_hbm,
                        idx1_v, idx2_v, r1_v, r2_v, sem1, sem2):
        cid = lax.axis_index("c")
        sid = lax.axis_index("s")
        wid = sid * 2 + cid
        tok0 = wid * tpw
        pltpu.sync_copy(pos_hbm.at[0, pl.ds(tok0, 128)], idx1_v)
        pltpu.sync_copy(pos_hbm.at[1, pl.ds(tok0, 128)], idx2_v)
        c1 = pltpu.async_copy(ex_hbm.at[idx1_v], r1_v, sem1)
        c2 = pltpu.async_copy(ex_hbm.at[idx2_v], r2_v, sem2)
        c1.wait()
        c2.wait()

        def add_body(t, _):
            for c in range(N_EMBD // 16):
                sl = pl.ds(c * 16, 16)
                r1_v[t, sl] = r1_v[t, sl] + r2_v[t, sl]
            return ()

        lax.fori_loop(0, 128, add_body, (), unroll=4)
        pltpu.sync_copy(r1_v, out_hbm.at[pl.ds(tok0, 128)])

    return _combine_kernel


def _half_pipeline(flat, Wr, W1p, W3p, W2p, mesh):
    nt, C = flat.shape
    n_ent = 2 * nt
    cap = n_ent + N_EXPERTS * TILE
    n_tiles = cap // TILE

    pos_pk, wn_pk, seg = pl.pallas_call(
        _make_plan_kernel(nt),
        in_specs=[
            pl.BlockSpec((nt, C), lambda: (0, 0)),
            pl.BlockSpec((N_EXPERTS, C), lambda: (0, 0)),
        ],
        out_specs=[
            pl.BlockSpec((1, n_ent), lambda: (0, 0)),
            pl.BlockSpec((1, n_ent), lambda: (0, 0)),
            pl.BlockSpec((N_EXPERTS, 2), lambda: (0, 0)),
        ],
        out_shape=[
            jax.ShapeDtypeStruct((1, n_ent), jnp.int32),
            jax.ShapeDtypeStruct((1, n_ent), jnp.float32),
            jax.ShapeDtypeStruct((N_EXPERTS, 2), jnp.int32),
        ],
    )(flat, Wr)

    pos2d = pos_pk.reshape(n_ent // 128, 128)
    wn2d = wn_pk.reshape(n_ent // 128, 128)
    pos = pos_pk.reshape(2, nt)

    seg_end = seg[:, 1]
    tile_base = jnp.arange(n_tiles, dtype=jnp.int32) * TILE
    te = jnp.sum((tile_base[:, None] >= seg_end[None, :]).astype(jnp.int32),
                 axis=1)
    te = jnp.clip(te, 0, N_EXPERTS - 1)

    xs, wrow = pl.kernel(
        _make_dispatch_kernel(nt),
        out_type=[
            jax.ShapeDtypeStruct((cap, C), jnp.float32),
            jax.ShapeDtypeStruct((cap,), jnp.float32),
        ],
        mesh=mesh,
        scratch_types=[
            pltpu.VMEM((2, 128), jnp.int32),
            pltpu.VMEM((2, 128), jnp.float32),
            pltpu.VMEM((nt // NW, C), jnp.float32),
            pltpu.SemaphoreType.DMA,
        ],
        compiler_params=pltpu.CompilerParams(needs_layout_passes=False),
    )(pos2d, wn2d, flat)

    grid_spec = pltpu.PrefetchScalarGridSpec(
        num_scalar_prefetch=1,
        grid=(n_tiles,),
        in_specs=[
            pl.BlockSpec((TILE, C), lambda i, te_ref: (i, 0)),
            pl.BlockSpec((1, 1, TILE), lambda i, te_ref: (i, 0, 0)),
            pl.BlockSpec((1, HID_PAD, C),
                         lambda i, te_ref: (te_ref[i], 0, 0)),
            pl.BlockSpec((1, HID_PAD, C),
                         lambda i, te_ref: (te_ref[i], 0, 0)),
            pl.BlockSpec((1, C, HID_PAD),
                         lambda i, te_ref: (te_ref[i], 0, 0)),
        ],
        out_specs=pl.BlockSpec((TILE, C), lambda i, te_ref: (i, 0)),
    )
    ex = pl.pallas_call(
        _expert_kernel,
        grid_spec=grid_spec,
        out_shape=jax.ShapeDtypeStruct((cap, C), jnp.float32),
        compiler_params=pltpu.CompilerParams(
            dimension_semantics=("arbitrary",),
        ),
    )(te, xs, wrow.reshape(n_tiles, 1, TILE), W1p, W3p, W2p)

    out = pl.kernel(
        _make_combine_kernel(nt),
        out_type=jax.ShapeDtypeStruct((nt, C), jnp.float32),
        mesh=mesh,
        scratch_types=[
            pltpu.VMEM((128,), jnp.int32),
            pltpu.VMEM((128,), jnp.int32),
            pltpu.VMEM((128, C), jnp.float32),
            pltpu.VMEM((128, C), jnp.float32),
            pltpu.SemaphoreType.DMA,
            pltpu.SemaphoreType.DMA,
        ],
        compiler_params=pltpu.CompilerParams(needs_layout_passes=False),
    )(pos, ex)
    return out


def kernel(x, W1, W2, W3, Wr):
    B, T, C = x.shape
    flat = x.reshape(-1, C)
    n_tok = flat.shape[0]
    pad = HID_PAD - HIDDEN
    W1p = jnp.pad(W1, ((0, 0), (0, pad), (0, 0))).astype(jnp.bfloat16)
    W3p = jnp.pad(W3, ((0, 0), (0, pad), (0, 0))).astype(jnp.bfloat16)
    W2p = jnp.pad(W2, ((0, 0), (0, 0), (0, pad))).astype(jnp.bfloat16)

    mesh = plsc.VectorSubcoreMesh(core_axis_name="c", subcore_axis_name="s",
                                  num_cores=2, num_subcores=16)
    half = n_tok // 2
    out0 = _half_pipeline(flat[:half], Wr, W1p, W3p, W2p, mesh)
    out1 = _half_pipeline(flat[half:], Wr, W1p, W3p, W2p, mesh)
    return jnp.concatenate([out0, out1], axis=0).reshape(B, T, C)


# speedup vs baseline: 1.0398x; 1.0398x over previous
import functools

import jax
import jax.numpy as jnp
from jax import lax
from jax.experimental import pallas as pl
from jax.experimental.pallas import tpu as pltpu
from jax.experimental.pallas import tpu_sc as plsc

N_EMBD = 256
N_EXPERTS = 8
HIDDEN = 682
HID_PAD = 768
N_TOK = 8192
N_ENT = 2 * N_TOK
TILE = 512
CAP = N_ENT + N_EXPERTS * TILE
N_TILES = CAP // TILE
WIN = 128
N_WIN = CAP // WIN
NW = 32
NSUB = 16
EPS = 1024


def _shift_rows(a, k):
    return jnp.pad(a, ((k, 0), (0, 0)))[: a.shape[0], :]


def _shift_lanes(a, k):
    return jnp.pad(a, ((0, 0), (k, 0)))[:, : a.shape[1]]


def _plan_kernel(x_ref, wr_ref, pos_ref, wn_ref, seg_ref):
    x = x_ref[...]
    logits_t = lax.dot_general(
        wr_ref[...], x, (((1,), (1,)), ((), ())),
        preferred_element_type=jnp.float32)
    m = jnp.max(logits_t, axis=0, keepdims=True)
    unnorm = jnp.exp(logits_t - m)
    probs = unnorm / jnp.sum(unnorm, axis=0, keepdims=True)
    eidx = lax.broadcasted_iota(jnp.int32, probs.shape, 0)
    p1 = jnp.max(probs, axis=0, keepdims=True)
    i1 = jnp.min(jnp.where(probs == p1, eidx, N_EXPERTS), axis=0,
                 keepdims=True)
    probs_m = jnp.where(eidx == i1, -1.0, probs)
    p2 = jnp.max(probs_m, axis=0, keepdims=True)
    i2 = jnp.min(jnp.where(probs_m == p2, eidx, N_EXPERTS), axis=0,
                 keepdims=True)
    denom = p1 + p2 + 1e-9
    w1n = p1 / denom
    w2n = p2 / denom

    oh1 = (eidx == i1).astype(jnp.float32)
    oh2 = (eidx == i2).astype(jnp.float32)
    oht = jnp.concatenate([oh1, oh2], axis=1)

    inc = oht
    k = 1
    while k < N_ENT:
        inc = inc + _shift_lanes(inc, k)
        k *= 2
    excl = inc - oht

    counts_i = inc[:, N_ENT - 1:N_ENT].astype(jnp.int32)
    padded = ((counts_i + TILE - 1) // TILE) * TILE
    t = padded
    for k in (1, 2, 4):
        t = t + _shift_rows(t, k)
    seg_start = t - padded
    seg_end = t

    rank = jnp.sum(excl * oht, axis=0, keepdims=True)
    base = jnp.sum(oht * seg_start.astype(jnp.float32), axis=0,
                   keepdims=True)
    pos_ref[...] = (rank + base).astype(jnp.int32)
    wn_ref[...] = jnp.concatenate([w1n, w2n], axis=1)
    seg_ref[...] = jnp.concatenate([seg_start, seg_end], axis=1)


def _dispatch_kernel(pos_hbm, wn_hbm, flat_hbm, xs_hbm, wrow_hbm,
                     idx_v, wnv_v, rows_v, sem):
    cid = lax.axis_index("c")
    sid = lax.axis_index("s")
    wid = sid * 2 + cid
    tpw = N_TOK // NW

    pltpu.sync_copy(flat_hbm.at[pl.ds(wid * tpw, tpw)], rows_v)
    pltpu.sync_copy(pos_hbm.at[pl.ds(2 * wid, 2)], idx_v.at[pl.ds(0, 2)])
    pltpu.sync_copy(pos_hbm.at[pl.ds(N_TOK // 128 + 2 * wid, 2)],
                    idx_v.at[pl.ds(2, 2)])
    pltpu.sync_copy(wn_hbm.at[pl.ds(2 * wid, 2)], wnv_v.at[pl.ds(0, 2)])
    pltpu.sync_copy(wn_hbm.at[pl.ds(N_TOK // 128 + 2 * wid, 2)],
                    wnv_v.at[pl.ds(2, 2)])

    cps = []
    for r0, irow in ((0, 0), (128, 1), (0, 2), (128, 3)):
        cps.append(pltpu.async_copy(
            rows_v.at[pl.ds(r0, 128)], xs_hbm.at[idx_v.at[irow]], sem))
    for cp in cps:
        cp.wait()
    for irow in range(4):
        pltpu.sync_copy(wnv_v.at[irow], wrow_hbm.at[idx_v.at[irow]])


def _expert_kernel(te_ref, xs_ref, wrow_ref, w1_ref, w3_ref, w2_ref,
                   ex_ref):
    xb = xs_ref[...].astype(jnp.bfloat16)
    g = lax.dot_general(
        xb, w1_ref[0], (((1,), (1,)), ((), ())),
        preferred_element_type=jnp.float32)
    u = lax.dot_general(
        xb, w3_ref[0], (((1,), (1,)), ((), ())),
        preferred_element_type=jnp.float32)
    h = (g * jax.nn.sigmoid(g)) * u
    ex = lax.dot_general(
        h.astype(jnp.bfloat16), w2_ref[0], (((1,), (1,)), ((), ())),
        preferred_element_type=jnp.float32)
    ex_ref[...] = ex * wrow_ref[0, 0][:, None]


def _combine_kernel(pos_hbm, ex_hbm, out_hbm,
                    idx1_v, idx2_v, r1_v, r2_v, sem1, sem2):
    cid = lax.axis_index("c")
    sid = lax.axis_index("s")
    wid = sid * 2 + cid
    for chunk in range(2):
        tok0 = wid * (N_TOK // NW) + chunk * 128
        pltpu.sync_copy(pos_hbm.at[0, pl.ds(tok0, 128)], idx1_v)
        pltpu.sync_copy(pos_hbm.at[1, pl.ds(tok0, 128)], idx2_v)
        c1 = pltpu.async_copy(ex_hbm.at[idx1_v], r1_v, sem1)
        c2 = pltpu.async_copy(ex_hbm.at[idx2_v], r2_v, sem2)
        c1.wait()
        c2.wait()

        def add_body(t, _):
            for c in range(N_EMBD // 16):
                sl = pl.ds(c * 16, 16)
                r1_v[t, sl] = r1_v[t, sl] + r2_v[t, sl]
            return ()

        lax.fori_loop(0, 128, add_body, (), unroll=4)
        pltpu.sync_copy(r1_v, out_hbm.at[pl.ds(tok0, 128)])


def kernel(x, W1, W2, W3, Wr):
    B, T, C = x.shape
    flat = x.reshape(-1, C)
    pad = HID_PAD - HIDDEN
    W1p = jnp.pad(W1, ((0, 0), (0, pad), (0, 0))).astype(jnp.bfloat16)
    W3p = jnp.pad(W3, ((0, 0), (0, pad), (0, 0))).astype(jnp.bfloat16)
    W2p = jnp.pad(W2, ((0, 0), (0, 0), (0, pad))).astype(jnp.bfloat16)

    pos_pk, wn_pk, seg = pl.pallas_call(
        _plan_kernel,
        in_specs=[
            pl.BlockSpec((N_TOK, C), lambda: (0, 0)),
            pl.BlockSpec((N_EXPERTS, C), lambda: (0, 0)),
        ],
        out_specs=[
            pl.BlockSpec((1, N_ENT), lambda: (0, 0)),
            pl.BlockSpec((1, N_ENT), lambda: (0, 0)),
            pl.BlockSpec((N_EXPERTS, 2), lambda: (0, 0)),
        ],
        out_shape=[
            jax.ShapeDtypeStruct((1, N_ENT), jnp.int32),
            jax.ShapeDtypeStruct((1, N_ENT), jnp.float32),
            jax.ShapeDtypeStruct((N_EXPERTS, 2), jnp.int32),
        ],
    )(flat, Wr)

    pos2d = pos_pk.reshape(N_ENT // 128, 128)
    wn2d = wn_pk.reshape(N_ENT // 128, 128)
    pos = pos_pk.reshape(2, N_TOK)

    seg_end = seg[:, 1]
    tile_base = jnp.arange(N_TILES, dtype=jnp.int32) * TILE
    te = jnp.sum((tile_base[:, None] >= seg_end[None, :]).astype(jnp.int32),
                 axis=1)
    te = jnp.clip(te, 0, N_EXPERTS - 1)

    mesh = plsc.VectorSubcoreMesh(core_axis_name="c", subcore_axis_name="s",
                                  num_cores=2, num_subcores=16)
    xs, wrow = pl.kernel(
        _dispatch_kernel,
        out_type=[
            jax.ShapeDtypeStruct((CAP, C), jnp.float32),
            jax.ShapeDtypeStruct((CAP,), jnp.float32),
        ],
        mesh=mesh,
        scratch_types=[
            pltpu.VMEM((4, 128), jnp.int32),
            pltpu.VMEM((4, 128), jnp.float32),
            pltpu.VMEM((N_TOK // NW, C), jnp.float32),
            pltpu.SemaphoreType.DMA,
        ],
        compiler_params=pltpu.CompilerParams(needs_layout_passes=False),
    )(pos2d, wn2d, flat)

    grid_spec = pltpu.PrefetchScalarGridSpec(
        num_scalar_prefetch=1,
        grid=(N_TILES,),
        in_specs=[
            pl.BlockSpec((TILE, C), lambda i, te_ref: (i, 0)),
            pl.BlockSpec((1, 1, TILE), lambda i, te_ref: (i, 0, 0)),
            pl.BlockSpec((1, HID_PAD, C),
                         lambda i, te_ref: (te_ref[i], 0, 0)),
            pl.BlockSpec((1, HID_PAD, C),
                         lambda i, te_ref: (te_ref[i], 0, 0)),
            pl.BlockSpec((1, C, HID_PAD),
                         lambda i, te_ref: (te_ref[i], 0, 0)),
        ],
        out_specs=pl.BlockSpec((TILE, C), lambda i, te_ref: (i, 0)),
    )
    ex = pl.pallas_call(
        _expert_kernel,
        grid_spec=grid_spec,
        out_shape=jax.ShapeDtypeStruct((CAP, C), jnp.float32),
        compiler_params=pltpu.CompilerParams(
            dimension_semantics=("arbitrary",),
        ),
    )(te, xs, wrow.reshape(N_TILES, 1, TILE), W1p, W3p, W2p)

    out = pl.kernel(
        _combine_kernel,
        out_type=jax.ShapeDtypeStruct((N_TOK, C), jnp.float32),
        mesh=mesh,
        scratch_types=[
            pltpu.VMEM((128,), jnp.int32),
            pltpu.VMEM((128,), jnp.int32),
            pltpu.VMEM((128, C), jnp.float32),
            pltpu.VMEM((128, C), jnp.float32),
            pltpu.SemaphoreType.DMA,
            pltpu.SemaphoreType.DMA,
        ],
        compiler_params=pltpu.CompilerParams(needs_layout_passes=False),
    )(pos, ex)

    return out.reshape(B, T, C)
